# baseline (speedup 1.0000x reference)
import functools

import jax
import jax.numpy as jnp
from jax import lax
from jax.experimental import pallas as pl
from jax.experimental.pallas import tpu as pltpu
from jax.experimental.pallas import tpu_sc as plsc

BATCH = 16384
DIM = 64
LANES = 16
NUM_CORES = 2
NUM_SUBCORES = 16
NUM_WORKERS = NUM_CORES * NUM_SUBCORES
ROWS_PER_WORKER = BATCH // NUM_WORKERS
CHUNK = 128
NCHUNK = ROWS_PER_WORKER // CHUNK


def _dict_embed_kernel(user_hbm, item_hbm, ctx_hbm, wu_hbm, wi_hbm, wc_hbm,
                       out_hbm, idx_u, idx_i, idx_c, rows_u, rows_i, rows_c,
                       sem_u, sem_i, sem_c, sem_out):
    wid = lax.axis_index("s") * NUM_CORES + lax.axis_index("c")
    chunk_base = wid * NCHUNK
    row_base = wid * ROWS_PER_WORKER

    pltpu.sync_copy(user_hbm.at[pl.ds(chunk_base, NCHUNK)], idx_u)
    pltpu.sync_copy(item_hbm.at[pl.ds(chunk_base, NCHUNK)], idx_i)
    pltpu.sync_copy(ctx_hbm.at[pl.ds(chunk_base, NCHUNK)], idx_c)

    copies = []
    for j in range(NCHUNK):
        dst = rows_u.at[pl.ds(j * CHUNK, CHUNK)]
        copies.append(pltpu.async_copy(wu_hbm.at[idx_u.at[j]], dst, sem_u))
    for j in range(NCHUNK):
        dst = rows_i.at[pl.ds(j * CHUNK, CHUNK)]
        copies.append(pltpu.async_copy(wi_hbm.at[idx_i.at[j]], dst, sem_i))
    for j in range(NCHUNK):
        dst = rows_c.at[pl.ds(j * CHUNK, CHUNK)]
        copies.append(pltpu.async_copy(wc_hbm.at[idx_c.at[j]], dst, sem_c))
    for c in copies:
        c.wait()

    def body(r, _):
        for c in range(DIM // LANES):
            sl = pl.ds(c * LANES, LANES)
            rows_u[r, sl] = rows_u[r, sl] + rows_i[r, sl] + rows_c[r, sl]
        return _

    lax.fori_loop(0, ROWS_PER_WORKER, body, 0)

    pltpu.async_copy(rows_u, out_hbm.at[pl.ds(row_base, ROWS_PER_WORKER)],
                     sem_out).wait()


@jax.jit
def _dict_embed(user2d, item2d, ctx2d, wu, wi, wc):
    mesh = plsc.VectorSubcoreMesh(core_axis_name="c", subcore_axis_name="s")
    return pl.kernel(
        _dict_embed_kernel,
        mesh=mesh,
        out_type=jax.ShapeDtypeStruct((BATCH, DIM), jnp.float32),
        scratch_types=[
            pltpu.VMEM((NCHUNK, CHUNK), jnp.int32),
            pltpu.VMEM((NCHUNK, CHUNK), jnp.int32),
            pltpu.VMEM((NCHUNK, CHUNK), jnp.int32),
            pltpu.VMEM((ROWS_PER_WORKER, DIM), jnp.float32),
            pltpu.VMEM((ROWS_PER_WORKER, DIM), jnp.float32),
            pltpu.VMEM((ROWS_PER_WORKER, DIM), jnp.float32),
            pltpu.SemaphoreType.DMA,
            pltpu.SemaphoreType.DMA,
            pltpu.SemaphoreType.DMA,
            pltpu.SemaphoreType.DMA,
        ],
        compiler_params=pltpu.CompilerParams(use_tc_tiling_on_sc=False),
    )(user2d, item2d, ctx2d, wu, wi, wc)


def kernel(user, item, context, W_user, W_item, W_context):
    user2d = user.astype(jnp.int32).reshape(BATCH // CHUNK, CHUNK)
    item2d = item.astype(jnp.int32).reshape(BATCH // CHUNK, CHUNK)
    ctx2d = context.astype(jnp.int32).reshape(BATCH // CHUNK, CHUNK)
    return _dict_embed(user2d, item2d, ctx2d, W_user, W_item, W_context)
